# baseline (speedup 1.0000x reference)
import functools

import jax
import jax.numpy as jnp
from jax import lax
from jax.experimental import pallas as pl
from jax.experimental.pallas import tpu as pltpu
from jax.experimental.pallas import tpu_sc as plsc

N = 100000
E = 1600000
G = 256
Z = 10

NC = 2
NS = 16
NW = NC * NS

ES = 50048
ECH = 2176
NCH = ES // ECH
UNROLL = 8
EBODIES = ECH // (16 * UNROLL)

NODES_W = 3136
NBLK = NODES_W // 16

_mesh = plsc.VectorSubcoreMesh(core_axis_name="c", subcore_axis_name="s")
_params = pltpu.CompilerParams(needs_layout_passes=False,
                               use_tc_tiling_on_sc=False)


@functools.partial(
    pl.kernel,
    mesh=_mesh,
    compiler_params=_params,
    out_type=jax.ShapeDtypeStruct((NW * G,), jnp.float32),
    scratch_types=[
        pltpu.VMEM((N,), jnp.int32),
        pltpu.VMEM((2, ECH), jnp.int32),
        pltpu.VMEM((2, ECH), jnp.int32),
        pltpu.VMEM((G,), jnp.float32),
        pltpu.SemaphoreType.DMA,
        pltpu.SemaphoreType.DMA,
        pltpu.SemaphoreType.DMA,
    ],
)
def _edge_flags(edge_index, batch_hbm, flags_out, batch_v, dst_a, dst_b,
                flag_v, sem_a, sem_b, sem_t):
    wid = lax.axis_index("s") * NC + lax.axis_index("c")
    base = jnp.minimum(wid * ES, E - ES)
    tbl_cp = pltpu.async_copy(batch_hbm, batch_v, sem_t)
    bufs = (dst_a, dst_b)
    sems = (sem_a, sem_b)
    cps = [None, None]
    cps[0] = pltpu.async_copy(edge_index.at[:, pl.ds(base, ECH)],
                              bufs[0], sems[0])
    zero16 = jnp.zeros((16,), jnp.float32)
    for b in range(G // 16):
        flag_v[pl.ds(b * 16, 16)] = zero16
    one16 = jnp.ones((16,), jnp.float32)
    tbl_cp.wait()

    for k in range(NCH):
        if k + 1 < NCH:
            cps[(k + 1) % 2] = pltpu.async_copy(
                edge_index.at[:, pl.ds(base + (k + 1) * ECH, ECH)],
                bufs[(k + 1) % 2], sems[(k + 1) % 2])
        cps[k % 2].wait()
        dst_v = bufs[k % 2]

        def body(i, carry, dst_v=dst_v):
            for u in range(UNROLL):
                off = (i * UNROLL + u) * 16
                idx16 = dst_v[1, pl.ds(off, 16)]
                g16 = plsc.load_gather(batch_v, [idx16])
                plsc.store_scatter(flag_v, [g16], one16)
            return carry

        lax.fori_loop(0, EBODIES, body, 0)

    pltpu.sync_copy(flag_v, flags_out.at[pl.ds(wid * G, G)])


@functools.partial(
    pl.kernel,
    mesh=_mesh,
    compiler_params=_params,
    out_type=jax.ShapeDtypeStruct((N,), jnp.float32),
    scratch_types=[
        pltpu.VMEM((NW * G,), jnp.float32),
        pltpu.VMEM((G + 1,), jnp.int32),
        pltpu.VMEM((G,), jnp.float32),
        pltpu.VMEM((2, Z), jnp.float32),
        pltpu.VMEM((2, Z), jnp.float32),
        pltpu.VMEM((NODES_W,), jnp.float32),
        pltpu.VMEM((NODES_W,), jnp.int32),
        pltpu.VMEM((NODES_W,), jnp.int32),
        pltpu.VMEM((NODES_W, Z), jnp.float32),
        pltpu.VMEM((NODES_W,), jnp.float32),
    ],
)
def _node_energy(flags_hbm, ptr_hbm, energy_hbm, batch_hbm,
                 level_hbm, attrs_hbm, scale_hbm, shift_hbm, out_hbm,
                 flags_v, ptr_v, mask_v, sc_v, sh_v,
                 en_v, bat_v, lev_v, att_v, res_v):
    wid = lax.axis_index("s") * NC + lax.axis_index("c")
    base = jnp.minimum(wid * NODES_W, N - NODES_W)
    pltpu.sync_copy(flags_hbm, flags_v)
    pltpu.sync_copy(ptr_hbm, ptr_v)
    pltpu.sync_copy(scale_hbm, sc_v)
    pltpu.sync_copy(shift_hbm, sh_v)
    pltpu.sync_copy(energy_hbm.at[pl.ds(base, NODES_W)], en_v)
    pltpu.sync_copy(batch_hbm.at[pl.ds(base, NODES_W)], bat_v)
    pltpu.sync_copy(level_hbm.at[pl.ds(base, NODES_W)], lev_v)
    pltpu.sync_copy(attrs_hbm.at[pl.ds(base, NODES_W), :], att_v)

    zero16 = jnp.zeros((16,), jnp.float32)
    for b in range(G // 16):
        off = b * 16

        def racc(r, acc, off=off):
            return acc + flags_v[pl.ds(r * G + off, 16)]

        edges = lax.fori_loop(0, NW, racc, zero16)
        iota16 = lax.iota(jnp.int32, 16)
        nn = plsc.load_gather(ptr_v, [iota16 + (off + 1)]) - ptr_v[pl.ds(off, 16)]
        iso = jnp.logical_and(nn == 1, edges == 0.0)
        mask_v[pl.ds(off, 16)] = jnp.where(iso, 0.0, 1.0)

    zi = jnp.zeros((16,), jnp.int32)
    iota = lax.iota(jnp.int32, 16)

    def nblock(j, carry):
        off = j * 16
        e16 = en_v[pl.ds(off, 16)]
        b16 = bat_v[pl.ds(off, 16)]
        l16 = lev_v[pl.ds(off, 16)]
        m16 = plsc.load_gather(mask_v, [b16])
        row16 = iota + off
        s = zero16
        t = zero16
        for z in range(Z):
            a_z = plsc.load_gather(att_v, [row16, zi + z])
            s = s + a_z * plsc.load_gather(sc_v, [l16, zi + z])
            t = t + a_z * plsc.load_gather(sh_v, [l16, zi + z])
        res_v[pl.ds(off, 16)] = m16 * (e16 * s + t)
        return carry

    lax.fori_loop(0, NBLK, nblock, 0)
    pltpu.sync_copy(res_v, out_hbm.at[pl.ds(base, NODES_W)])


def kernel(node_energy, node_attrs, ptr, edge_index, batch, node_level,
           scale, shift):
    flags = _edge_flags(edge_index, batch)
    return _node_energy(flags, ptr, node_energy, batch, node_level,
                        node_attrs, scale, shift)


# speedup vs baseline: 1.0139x; 1.0139x over previous
import functools

import jax
import jax.numpy as jnp
from jax import lax
from jax.experimental import pallas as pl
from jax.experimental.pallas import tpu as pltpu
from jax.experimental.pallas import tpu_sc as plsc

N = 100000
E = 1600000
G = 256
Z = 10

NC = 2
NS = 16
NW = NC * NS

ES = 50048
ECH = 2176
NCH = ES // ECH
UNROLL = 8
EBODIES = ECH // (16 * UNROLL)

NODES_W = 3136
NBLK = NODES_W // 16

_mesh = plsc.VectorSubcoreMesh(core_axis_name="c", subcore_axis_name="s")
_params = pltpu.CompilerParams(needs_layout_passes=False,
                               use_tc_tiling_on_sc=False)


@functools.partial(
    pl.kernel,
    mesh=_mesh,
    compiler_params=_params,
    out_type=jax.ShapeDtypeStruct((NW * G,), jnp.float32),
    scratch_types=[
        pltpu.VMEM((N,), jnp.int32),
        pltpu.VMEM((2, ECH), jnp.int32),
        pltpu.VMEM((2, ECH), jnp.int32),
        pltpu.VMEM((G,), jnp.float32),
        pltpu.SemaphoreType.DMA,
        pltpu.SemaphoreType.DMA,
        pltpu.SemaphoreType.DMA,
    ],
)
def _edge_flags(edge_index, batch_hbm, flags_out, batch_v, dst_a, dst_b,
                flag_v, sem_a, sem_b, sem_t):
    wid = lax.axis_index("s") * NC + lax.axis_index("c")
    base = jnp.minimum(wid * ES, E - ES)
    tbl_cp = pltpu.async_copy(batch_hbm, batch_v, sem_t)
    bufs = (dst_a, dst_b)
    sems = (sem_a, sem_b)
    cps = [None, None]
    cps[0] = pltpu.async_copy(edge_index.at[:, pl.ds(base, ECH)],
                              bufs[0], sems[0])
    zero16 = jnp.zeros((16,), jnp.float32)
    for b in range(G // 16):
        flag_v[pl.ds(b * 16, 16)] = zero16
    one16 = jnp.ones((16,), jnp.float32)
    tbl_cp.wait()

    for k in range(NCH):
        if k + 1 < NCH:
            cps[(k + 1) % 2] = pltpu.async_copy(
                edge_index.at[:, pl.ds(base + (k + 1) * ECH, ECH)],
                bufs[(k + 1) % 2], sems[(k + 1) % 2])
        cps[k % 2].wait()
        dst_v = bufs[k % 2]

        @plsc.parallel_loop(0, ECH // 16, 1, unroll=UNROLL)
        def body(i, dst_v=dst_v):
            off = i * 16
            idx16 = dst_v[1, pl.ds(off, 16)]
            g16 = plsc.load_gather(batch_v, [idx16])
            plsc.store_scatter(flag_v, [g16], one16)

    pltpu.sync_copy(flag_v, flags_out.at[pl.ds(wid * G, G)])


@functools.partial(
    pl.kernel,
    mesh=_mesh,
    compiler_params=_params,
    out_type=jax.ShapeDtypeStruct((N,), jnp.float32),
    scratch_types=[
        pltpu.VMEM((NW * G,), jnp.float32),
        pltpu.VMEM((G + 1,), jnp.int32),
        pltpu.VMEM((G,), jnp.float32),
        pltpu.VMEM((2, Z), jnp.float32),
        pltpu.VMEM((2, Z), jnp.float32),
        pltpu.VMEM((NODES_W,), jnp.float32),
        pltpu.VMEM((NODES_W,), jnp.int32),
        pltpu.VMEM((NODES_W,), jnp.int32),
        pltpu.VMEM((NODES_W, Z), jnp.float32),
        pltpu.VMEM((NODES_W,), jnp.float32),
    ],
)
def _node_energy(flags_hbm, ptr_hbm, energy_hbm, batch_hbm,
                 level_hbm, attrs_hbm, scale_hbm, shift_hbm, out_hbm,
                 flags_v, ptr_v, mask_v, sc_v, sh_v,
                 en_v, bat_v, lev_v, att_v, res_v):
    wid = lax.axis_index("s") * NC + lax.axis_index("c")
    base = jnp.minimum(wid * NODES_W, N - NODES_W)
    pltpu.sync_copy(flags_hbm, flags_v)
    pltpu.sync_copy(ptr_hbm, ptr_v)
    pltpu.sync_copy(scale_hbm, sc_v)
    pltpu.sync_copy(shift_hbm, sh_v)
    pltpu.sync_copy(energy_hbm.at[pl.ds(base, NODES_W)], en_v)
    pltpu.sync_copy(batch_hbm.at[pl.ds(base, NODES_W)], bat_v)
    pltpu.sync_copy(level_hbm.at[pl.ds(base, NODES_W)], lev_v)
    pltpu.sync_copy(attrs_hbm.at[pl.ds(base, NODES_W), :], att_v)

    zero16 = jnp.zeros((16,), jnp.float32)
    for b in range(G // 16):
        off = b * 16

        def racc(r, acc, off=off):
            return acc + flags_v[pl.ds(r * G + off, 16)]

        edges = lax.fori_loop(0, NW, racc, zero16)
        iota16 = lax.iota(jnp.int32, 16)
        nn = plsc.load_gather(ptr_v, [iota16 + (off + 1)]) - ptr_v[pl.ds(off, 16)]
        iso = jnp.logical_and(nn == 1, edges == 0.0)
        mask_v[pl.ds(off, 16)] = jnp.where(iso, 0.0, 1.0)

    zi = jnp.zeros((16,), jnp.int32)
    iota = lax.iota(jnp.int32, 16)

    @plsc.parallel_loop(0, NBLK, 1, unroll=2)
    def nblock(j):
        off = j * 16
        e16 = en_v[pl.ds(off, 16)]
        b16 = bat_v[pl.ds(off, 16)]
        l16 = lev_v[pl.ds(off, 16)]
        m16 = plsc.load_gather(mask_v, [b16])
        row16 = iota + off
        s = zero16
        t = zero16
        for z in range(Z):
            a_z = plsc.load_gather(att_v, [row16, zi + z])
            s = s + a_z * plsc.load_gather(sc_v, [l16, zi + z])
            t = t + a_z * plsc.load_gather(sh_v, [l16, zi + z])
        res_v[pl.ds(off, 16)] = m16 * (e16 * s + t)
    pltpu.sync_copy(res_v, out_hbm.at[pl.ds(base, NODES_W)])


def kernel(node_energy, node_attrs, ptr, edge_index, batch, node_level,
           scale, shift):
    flags = _edge_flags(edge_index, batch)
    return _node_energy(flags, ptr, node_energy, batch, node_level,
                        node_attrs, scale, shift)


# speedup vs baseline: 1.2290x; 1.2122x over previous
import functools

import jax
import jax.numpy as jnp
from jax import lax
from jax.experimental import pallas as pl
from jax.experimental.pallas import tpu as pltpu
from jax.experimental.pallas import tpu_sc as plsc

N = 100000
E = 1600000
G = 256
Z = 10

NC = 2
NS = 16
NW = NC * NS

ES = 50048
ECH = 2176
NCH = ES // ECH
UNROLL = 8

NODES_W = 3136
NBLK = NODES_W // 16

BLK = 8192

_mesh = plsc.VectorSubcoreMesh(core_axis_name="c", subcore_axis_name="s")
_params_edge = pltpu.CompilerParams(needs_layout_passes=False)
_params_sc = pltpu.CompilerParams(needs_layout_passes=False,
                                  use_tc_tiling_on_sc=False)


@functools.partial(
    pl.kernel,
    mesh=_mesh,
    compiler_params=_params_edge,
    out_type=jax.ShapeDtypeStruct((NW * G,), jnp.float32),
    scratch_types=[
        pltpu.VMEM((N,), jnp.int32),
        pltpu.VMEM((2, ECH), jnp.int32),
        pltpu.VMEM((2, ECH), jnp.int32),
        pltpu.VMEM((G,), jnp.float32),
        pltpu.SemaphoreType.DMA,
        pltpu.SemaphoreType.DMA,
        pltpu.SemaphoreType.DMA,
    ],
)
def _edge_flags(edge_index, batch_hbm, flags_out, batch_v, dst_a, dst_b,
                flag_v, sem_a, sem_b, sem_t):
    wid = lax.axis_index("s") * NC + lax.axis_index("c")
    base = jnp.minimum(wid * ES, E - ES)
    tbl_cp = pltpu.async_copy(batch_hbm, batch_v, sem_t)
    bufs = (dst_a, dst_b)
    sems = (sem_a, sem_b)
    cps = [None, None]
    cps[0] = pltpu.async_copy(edge_index.at[:, pl.ds(base, ECH)],
                              bufs[0], sems[0])
    zero16 = jnp.zeros((16,), jnp.float32)
    for b in range(G // 16):
        flag_v[pl.ds(b * 16, 16)] = zero16
    one16 = jnp.ones((16,), jnp.float32)
    tbl_cp.wait()

    for k in range(NCH):
        if k + 1 < NCH:
            cps[(k + 1) % 2] = pltpu.async_copy(
                edge_index.at[:, pl.ds(base + (k + 1) * ECH, ECH)],
                bufs[(k + 1) % 2], sems[(k + 1) % 2])
        cps[k % 2].wait()
        dst_v = bufs[k % 2]

        @plsc.parallel_loop(0, ECH // 16, 1, unroll=UNROLL)
        def body(i, dst_v=dst_v):
            off = i * 16
            idx16 = dst_v[1, pl.ds(off, 16)]
            g16 = plsc.load_gather(batch_v, [idx16])
            plsc.store_scatter(flag_v, [g16], one16)

    pltpu.sync_copy(flag_v, flags_out.at[pl.ds(wid * G, G)])


@functools.partial(
    pl.kernel,
    mesh=_mesh,
    compiler_params=_params_sc,
    out_type=jax.ShapeDtypeStruct((N,), jnp.float32),
    scratch_types=[
        pltpu.VMEM((NW * G,), jnp.float32),
        pltpu.VMEM((G + 1,), jnp.int32),
        pltpu.VMEM((G,), jnp.float32),
        pltpu.VMEM((2, Z), jnp.float32),
        pltpu.VMEM((2, Z), jnp.float32),
        pltpu.VMEM((NODES_W,), jnp.float32),
        pltpu.VMEM((NODES_W,), jnp.int32),
        pltpu.VMEM((NODES_W,), jnp.int32),
        pltpu.VMEM((NODES_W, Z), jnp.float32),
        pltpu.VMEM((NODES_W,), jnp.float32),
    ],
)
def _node_energy(flags_hbm, ptr_hbm, energy_hbm, batch_hbm,
                 level_hbm, attrs_hbm, scale_hbm, shift_hbm, out_hbm,
                 flags_v, ptr_v, mask_v, sc_v, sh_v,
                 en_v, bat_v, lev_v, att_v, res_v):
    wid = lax.axis_index("s") * NC + lax.axis_index("c")
    base = jnp.minimum(wid * NODES_W, N - NODES_W)
    pltpu.sync_copy(flags_hbm, flags_v)
    pltpu.sync_copy(ptr_hbm, ptr_v)
    pltpu.sync_copy(scale_hbm, sc_v)
    pltpu.sync_copy(shift_hbm, sh_v)
    pltpu.sync_copy(energy_hbm.at[pl.ds(base, NODES_W)], en_v)
    pltpu.sync_copy(batch_hbm.at[pl.ds(base, NODES_W)], bat_v)
    pltpu.sync_copy(level_hbm.at[pl.ds(base, NODES_W)], lev_v)
    pltpu.sync_copy(attrs_hbm.at[pl.ds(base, NODES_W), :], att_v)

    zero16 = jnp.zeros((16,), jnp.float32)
    iota = lax.iota(jnp.int32, 16)
    for b in range(G // 16):
        off = b * 16

        def racc(r, acc, off=off):
            return acc + flags_v[pl.ds(r * G + off, 16)]

        edges = lax.fori_loop(0, NW, racc, zero16)
        nn = plsc.load_gather(ptr_v, [iota + (off + 1)]) - ptr_v[pl.ds(off, 16)]
        iso = jnp.logical_and(nn == 1, edges == 0.0)
        mask_v[pl.ds(off, 16)] = jnp.where(iso, 0.0, 1.0)

    zi = jnp.zeros((16,), jnp.int32)

    @plsc.parallel_loop(0, NBLK, 1, unroll=2)
    def nblock(j):
        off = j * 16
        e16 = en_v[pl.ds(off, 16)]
        b16 = bat_v[pl.ds(off, 16)]
        l16 = lev_v[pl.ds(off, 16)]
        m16 = plsc.load_gather(mask_v, [b16])
        row16 = iota + off
        s = zero16
        t = zero16
        for z in range(Z):
            a_z = plsc.load_gather(att_v, [row16, zi + z])
            s = s + a_z * plsc.load_gather(sc_v, [l16, zi + z])
            t = t + a_z * plsc.load_gather(sh_v, [l16, zi + z])
        res_v[pl.ds(off, 16)] = m16 * (e16 * s + t)

    pltpu.sync_copy(res_v, out_hbm.at[pl.ds(base, NODES_W)])


def kernel(node_energy, node_attrs, ptr, edge_index, batch, node_level,
           scale, shift):
    flags = _edge_flags(edge_index, batch)
    return _node_energy(flags, ptr, node_energy, batch, node_level,
                        node_attrs, scale, shift)


# speedup vs baseline: 1.9330x; 1.5728x over previous
import functools

import jax
import jax.numpy as jnp
from jax import lax
from jax.experimental import pallas as pl
from jax.experimental.pallas import tpu as pltpu
from jax.experimental.pallas import tpu_sc as plsc

N = 100000
E = 1600000
G = 256
Z = 10

NC = 2
NS = 16
NW = NC * NS

ES = 50048
ECH = 2176
NCH = ES // ECH
UNROLL = 8

NODES_W = 3136
NBLK = NODES_W // 16

BLK = 8192

_mesh = plsc.VectorSubcoreMesh(core_axis_name="c", subcore_axis_name="s")
_params_edge = pltpu.CompilerParams(needs_layout_passes=False)
_params_sc = pltpu.CompilerParams(needs_layout_passes=False,
                                  use_tc_tiling_on_sc=False)


@functools.partial(
    pl.kernel,
    mesh=_mesh,
    compiler_params=_params_edge,
    out_type=jax.ShapeDtypeStruct((NW * G,), jnp.float32),
    scratch_types=[
        pltpu.VMEM((N,), jnp.int32),
        pltpu.VMEM((2, ECH), jnp.int32),
        pltpu.VMEM((2, ECH), jnp.int32),
        pltpu.VMEM((G,), jnp.float32),
        pltpu.SemaphoreType.DMA,
        pltpu.SemaphoreType.DMA,
        pltpu.SemaphoreType.DMA,
    ],
)
def _edge_flags(edge_index, batch_hbm, flags_out, batch_v, dst_a, dst_b,
                flag_v, sem_a, sem_b, sem_t):
    wid = lax.axis_index("s") * NC + lax.axis_index("c")
    base = jnp.minimum(wid * ES, E - ES)
    tbl_cp = pltpu.async_copy(batch_hbm, batch_v, sem_t)
    bufs = (dst_a, dst_b)
    sems = (sem_a, sem_b)
    cps = [None, None]
    cps[0] = pltpu.async_copy(edge_index.at[:, pl.ds(base, ECH)],
                              bufs[0], sems[0])
    zero16 = jnp.zeros((16,), jnp.float32)
    for b in range(G // 16):
        flag_v[pl.ds(b * 16, 16)] = zero16
    one16 = jnp.ones((16,), jnp.float32)
    tbl_cp.wait()

    for k in range(NCH):
        if k + 1 < NCH:
            cps[(k + 1) % 2] = pltpu.async_copy(
                edge_index.at[:, pl.ds(base + (k + 1) * ECH, ECH)],
                bufs[(k + 1) % 2], sems[(k + 1) % 2])
        cps[k % 2].wait()
        dst_v = bufs[k % 2]

        @plsc.parallel_loop(0, ECH // 16, 1, unroll=UNROLL)
        def body(i, dst_v=dst_v):
            off = i * 16
            idx16 = dst_v[1, pl.ds(off, 16)]
            g16 = plsc.load_gather(batch_v, [idx16])
            plsc.store_scatter(flag_v, [g16], one16)

    pltpu.sync_copy(flag_v, flags_out.at[pl.ds(wid * G, G)])


def _dense_body(w_ref, lf_ref, e_ref, a_ref, q_ref):
    c = lax.dot_general(w_ref[...], a_ref[...], (((0,), (1,)), ((), ())),
                        preferred_element_type=jnp.float32)
    lf = lf_ref[...]
    s = c[0:1, :] + lf * c[1:2, :]
    t = c[2:3, :] + lf * c[3:4, :]
    q_ref[...] = e_ref[...] * s + t


_dense = pl.pallas_call(
    _dense_body,
    grid=(pl.cdiv(N, BLK),),
    in_specs=[
        pl.BlockSpec((Z, 4), lambda i: (0, 0)),
        pl.BlockSpec((1, BLK), lambda i: (0, i)),
        pl.BlockSpec((1, BLK), lambda i: (0, i)),
        pl.BlockSpec((BLK, Z), lambda i: (i, 0)),
    ],
    out_specs=pl.BlockSpec((1, BLK), lambda i: (0, i)),
    out_shape=jax.ShapeDtypeStruct((1, N), jnp.float32),
)


@functools.partial(
    pl.kernel,
    mesh=_mesh,
    compiler_params=_params_sc,
    out_type=jax.ShapeDtypeStruct((N,), jnp.float32),
    scratch_types=[
        pltpu.VMEM((NW * G,), jnp.float32),
        pltpu.VMEM((G + 1,), jnp.int32),
        pltpu.VMEM((G,), jnp.float32),
        pltpu.VMEM((NODES_W,), jnp.int32),
        pltpu.VMEM((NODES_W,), jnp.float32),
        pltpu.VMEM((NODES_W,), jnp.float32),
    ],
)
def _mask_apply(flags_hbm, ptr_hbm, batch_hbm, q_hbm, out_hbm,
                flags_v, ptr_v, mask_v, bat_v, q_v, res_v):
    wid = lax.axis_index("s") * NC + lax.axis_index("c")
    base = jnp.minimum(wid * NODES_W, N - NODES_W)
    pltpu.sync_copy(flags_hbm, flags_v)
    pltpu.sync_copy(ptr_hbm, ptr_v)
    pltpu.sync_copy(batch_hbm.at[pl.ds(base, NODES_W)], bat_v)
    pltpu.sync_copy(q_hbm.at[pl.ds(base, NODES_W)], q_v)

    zero16 = jnp.zeros((16,), jnp.float32)
    iota = lax.iota(jnp.int32, 16)
    for b in range(G // 16):
        off = b * 16

        def racc(r, acc, off=off):
            return acc + flags_v[pl.ds(r * G + off, 16)]

        edges = lax.fori_loop(0, NW, racc, zero16)
        nn = plsc.load_gather(ptr_v, [iota + (off + 1)]) - ptr_v[pl.ds(off, 16)]
        iso = jnp.logical_and(nn == 1, edges == 0.0)
        mask_v[pl.ds(off, 16)] = jnp.where(iso, 0.0, 1.0)

    @plsc.parallel_loop(0, NBLK, 1, unroll=4)
    def nblock(j):
        off = j * 16
        b16 = bat_v[pl.ds(off, 16)]
        m16 = plsc.load_gather(mask_v, [b16])
        res_v[pl.ds(off, 16)] = m16 * q_v[pl.ds(off, 16)]

    pltpu.sync_copy(res_v, out_hbm.at[pl.ds(base, NODES_W)])


def kernel(node_energy, node_attrs, ptr, edge_index, batch, node_level,
           scale, shift):
    flags = _edge_flags(edge_index, batch)
    w = jnp.stack([scale[0], scale[1] - scale[0],
                   shift[0], shift[1] - shift[0]], axis=1)
    lf = node_level.astype(jnp.float32).reshape(1, N)
    e2 = node_energy.reshape(1, N)
    q = _dense(w, lf, e2, node_attrs).reshape(N)
    return _mask_apply(flags, ptr, batch, q)
